# Initial kernel scaffold; baseline (speedup 1.0000x reference)
import jax
import jax.numpy as jnp
from jax.experimental import pallas as pl


def kernel(x, edge_index, batch, W0, b0, W1, b1, W2, b2, W3, b3):
    raise NotImplementedError("write your pallas kernel here")


# speedup vs baseline: 53.3710x; 53.3710x over previous
import functools

import jax
import jax.numpy as jnp
from jax import lax
from jax.experimental import pallas as pl
from jax.experimental.pallas import tpu as pltpu
from jax.experimental.pallas import tpu_sc as plsc

N_NODES = 100000
N_EDGES = 6400000
NUM_GRAPHS = 512

NC = 2
NS = 16
NT = NC * NS

CH = 100
NCHUNKS = N_EDGES // CH
TILE_CHUNKS = NCHUNKS // NT
WIN = 8
NWIN = TILE_CHUNKS // WIN

NPAD = 100096
TPR = NPAD // NS

BROW = 2000
BP = 2000
NBP = N_NODES // BP

_mesh = lambda: plsc.VectorSubcoreMesh(core_axis_name="c", subcore_axis_name="s")
_SC_PARAMS = pltpu.CompilerParams(use_tc_tiling_on_sc=False)


def _sc_degree(dstr, zeros_t):
    @functools.partial(
        pl.kernel,
        out_type=jax.ShapeDtypeStruct((NC * NPAD,), jnp.float32),
        mesh=_mesh(),
        compiler_params=_SC_PARAMS,
        scratch_types=[
            pltpu.VMEM((WIN, CH), jnp.int32),
            pltpu.VMEM((112,), jnp.float32),
            pltpu.VMEM_SHARED((NPAD,), jnp.float32),
            pltpu.SemaphoreType.DMA,
            pltpu.SemaphoreType.DMA,
        ],
    )
    def degk(dst_hbm, z_hbm, out_hbm, dstw, ones_v, acc, sd, ss):
        cid = lax.axis_index("c")
        sid = lax.axis_index("s")
        tid = cid * NS + sid
        pltpu.sync_copy(z_hbm, acc.at[pl.ds(sid * TPR, TPR)])
        for j in range(7):
            ones_v[pl.ds(j * 16, 16)] = jnp.ones((16,), jnp.float32)
        plsc.subcore_barrier()

        @pl.loop(0, NWIN)
        def _(w):
            wbase = tid * TILE_CHUNKS + w * WIN
            pltpu.sync_copy(dst_hbm.at[pl.ds(wbase, WIN), :], dstw)
            hs = [
                pltpu.async_copy(
                    ones_v.at[pl.ds(0, CH)], acc.at[dstw.at[j]], ss, add=True
                )
                for j in range(WIN)
            ]
            for h in hs:
                h.wait()

        plsc.subcore_barrier()
        pltpu.sync_copy(
            acc.at[pl.ds(sid * TPR, TPR)],
            out_hbm.at[pl.ds(cid * NPAD + sid * TPR, TPR)],
        )

    return degk(dstr, zeros_t)


def _sc_aggregate(g, srcr, dstr, zeros_t, d):
    @functools.partial(
        pl.kernel,
        out_type=jax.ShapeDtypeStruct((NC, NPAD, d), jnp.float32),
        mesh=_mesh(),
        compiler_params=_SC_PARAMS,
        scratch_types=[
            pltpu.VMEM((WIN, CH), jnp.int32),
            pltpu.VMEM((WIN, CH), jnp.int32),
            pltpu.VMEM((WIN, CH, d), jnp.float32),
            pltpu.VMEM_SHARED((NPAD, d), jnp.float32),
            pltpu.SemaphoreType.DMA,
            pltpu.SemaphoreType.DMA,
            pltpu.SemaphoreType.DMA,
        ],
    )
    def aggk(g_hbm, src_hbm, dst_hbm, z_hbm, out_hbm, srcw, dstw, rows, acc, si, sg, ss):
        cid = lax.axis_index("c")
        sid = lax.axis_index("s")
        tid = cid * NS + sid
        pltpu.sync_copy(z_hbm, acc.at[pl.ds(sid * TPR, TPR), :])
        plsc.subcore_barrier()

        @pl.loop(0, NWIN)
        def _(w):
            wbase = tid * TILE_CHUNKS + w * WIN
            pltpu.sync_copy(src_hbm.at[pl.ds(wbase, WIN), :], srcw)
            pltpu.sync_copy(dst_hbm.at[pl.ds(wbase, WIN), :], dstw)
            ghs = [
                pltpu.async_copy(g_hbm.at[srcw.at[j]], rows.at[j], sg)
                for j in range(WIN)
            ]
            for h in ghs:
                h.wait()
            shs = [
                pltpu.async_copy(rows.at[j], acc.at[dstw.at[j]], ss, add=True)
                for j in range(WIN)
            ]
            for h in shs:
                h.wait()

        plsc.subcore_barrier()
        pltpu.sync_copy(
            acc.at[pl.ds(sid * TPR, TPR), :],
            out_hbm.at[cid, pl.ds(sid * TPR, TPR), :],
        )

    return aggk(g, srcr, dstr, zeros_t)


def _tc_prep(x, deg_t, w0, b0, w1):
    def body(x_ref, deg_ref, w0_ref, b0_ref, w1_ref, o_ref):
        deg = jnp.sum(deg_ref[...], axis=1, keepdims=True) + 1.0
        dis = lax.rsqrt(deg)
        x = x_ref[...]
        emb = x[:, 0:1] * w0_ref[...] + b0_ref[...]
        h0 = jnp.concatenate([emb, x[:, 1:4]], axis=1)
        w1v = w1_ref[...]
        h1 = h0[:, 0:1] * w1v[0:1, :]
        for k in range(1, 7):
            h1 = h1 + h0[:, k : k + 1] * w1v[k : k + 1, :]
        o_ref[...] = h1 * dis

    return pl.pallas_call(
        body,
        grid=(N_NODES // BROW,),
        in_specs=[
            pl.BlockSpec((BROW, 4), lambda i: (i, 0)),
            pl.BlockSpec((BROW, 2), lambda i: (i, 0)),
            pl.BlockSpec((1, 4), lambda i: (0, 0)),
            pl.BlockSpec((1, 4), lambda i: (0, 0)),
            pl.BlockSpec((7, 16), lambda i: (0, 0)),
        ],
        out_specs=pl.BlockSpec((BROW, 16), lambda i: (i, 0)),
        out_shape=jax.ShapeDtypeStruct((N_NODES, 16), jnp.float32),
    )(x, deg_t, w0, b0, w1)


def _tc_mid(s1p, g1, deg_t, w2, b1):
    def body(s_ref, g1_ref, deg_ref, w2_ref, b1_ref, o_ref):
        deg = jnp.sum(deg_ref[...], axis=1, keepdims=True) + 1.0
        dis = lax.rsqrt(deg)
        s = s_ref[0] + s_ref[1] + g1_ref[...]
        h1 = jnp.tanh(s * dis + b1_ref[...])
        w2v = w2_ref[...]
        h2 = h1[:, 0:1] * w2v[0:1, :]
        for k in range(1, 16):
            h2 = h2 + h1[:, k : k + 1] * w2v[k : k + 1, :]
        o_ref[...] = h2 * dis

    return pl.pallas_call(
        body,
        grid=(N_NODES // BROW,),
        in_specs=[
            pl.BlockSpec((2, BROW, 16), lambda i: (0, i, 0)),
            pl.BlockSpec((BROW, 16), lambda i: (i, 0)),
            pl.BlockSpec((BROW, 2), lambda i: (i, 0)),
            pl.BlockSpec((16, 8), lambda i: (0, 0)),
            pl.BlockSpec((1, 16), lambda i: (0, 0)),
        ],
        out_specs=pl.BlockSpec((BROW, 8), lambda i: (i, 0)),
        out_shape=jax.ShapeDtypeStruct((N_NODES, 8), jnp.float32),
    )(s1p, g1, deg_t, w2, b1)


def _tc_pool(s2p, g2, deg_t, batch3, b2, w3, b3):
    def body(s_ref, g2_ref, deg_ref, b_ref, b2_ref, w3_ref, b3_ref, o_ref, acc_ref):
        i = pl.program_id(0)

        @pl.when(i == 0)
        def _():
            acc_ref[...] = jnp.zeros_like(acc_ref)

        deg = jnp.sum(deg_ref[...], axis=1, keepdims=True) + 1.0
        dis = lax.rsqrt(deg)
        h2 = jnp.tanh((s_ref[0] + s_ref[1] + g2_ref[...]) * dis + b2_ref[...])
        h2e = jnp.concatenate(
            [h2, jnp.ones((BP, 1), jnp.float32), jnp.zeros((BP, 7), jnp.float32)],
            axis=1,
        )
        b = b_ref[0]
        gg = lax.broadcasted_iota(jnp.int32, (NUM_GRAPHS, BP), 0)
        oh = (b == gg).astype(jnp.float32)
        acc_ref[...] += jnp.dot(oh, h2e, preferred_element_type=jnp.float32)

        @pl.when(i == NBP - 1)
        def _():
            cnt = jnp.maximum(acc_ref[:, 8:9], 1.0)
            pooled = acc_ref[:, 0:8] / cnt
            o_ref[...] = (
                jnp.dot(pooled, w3_ref[...], preferred_element_type=jnp.float32)
                + b3_ref[...]
            )

    return pl.pallas_call(
        body,
        grid=(NBP,),
        in_specs=[
            pl.BlockSpec((2, BP, 8), lambda i: (0, i, 0)),
            pl.BlockSpec((BP, 8), lambda i: (i, 0)),
            pl.BlockSpec((BP, 2), lambda i: (i, 0)),
            pl.BlockSpec((1, 1, BP), lambda i: (i, 0, 0)),
            pl.BlockSpec((1, 8), lambda i: (0, 0)),
            pl.BlockSpec((8, 1), lambda i: (0, 0)),
            pl.BlockSpec((1, 1), lambda i: (0, 0)),
        ],
        out_specs=pl.BlockSpec((NUM_GRAPHS, 1), lambda i: (0, 0)),
        out_shape=jax.ShapeDtypeStruct((NUM_GRAPHS, 1), jnp.float32),
        scratch_shapes=[pltpu.VMEM((NUM_GRAPHS, 16), jnp.float32)],
    )(s2p, g2, deg_t, batch3, b2, w3, b3)


def kernel(x, edge_index, batch, W0, b0, W1, b1, W2, b2, W3, b3):
    srcr = edge_index[0].reshape(NCHUNKS, CH)
    dstr = edge_index[1].reshape(NCHUNKS, CH)
    batch3 = batch.reshape(NBP, 1, BP)
    z1 = jnp.zeros((TPR,), jnp.float32)
    z16 = jnp.zeros((TPR, 16), jnp.float32)
    z8 = jnp.zeros((TPR, 8), jnp.float32)
    w0r = W0.reshape(1, 4)
    b0r = b0.reshape(1, 4)
    b1r = b1.reshape(1, 16)
    b2r = b2.reshape(1, 8)
    b3r = b3.reshape(1, 1)

    degp = _sc_degree(dstr, z1).reshape(NC, NPAD)
    deg_t = jnp.swapaxes(degp, 0, 1)
    g1 = _tc_prep(x, deg_t, w0r, b0r, W1)
    s1p = _sc_aggregate(g1, srcr, dstr, z16, 16)
    g2 = _tc_mid(s1p, g1, deg_t, W2, b1r)
    s2p = _sc_aggregate(g2, srcr, dstr, z8, 8)
    return _tc_pool(s2p, g2, deg_t, batch3, b2r, W3, b3r)
